# baseline (speedup 1.0000x reference)
import functools

import jax
import jax.numpy as jnp
from jax import lax
from jax.experimental import pallas as pl
from jax.experimental.pallas import tpu as pltpu
from jax.experimental.pallas import tpu_sc as plsc

N = 10000
E = 320000
IN_CH = 128
HID = 64
W128 = 128
DW = 128

NC = 2
NS = 16
NW = NC * NS
CHUNK = 128
CPW = 80
EPW = CPW * CHUNK
EPAD = NW * EPW
RPT = 640
ACC_ROWS = NS * RPT
MB = 1000

_MESH = plsc.VectorSubcoreMesh(
    core_axis_name="c", subcore_axis_name="s", num_cores=NC, num_subcores=NS
)


@functools.partial(
    pl.kernel,
    out_type=jax.ShapeDtypeStruct((NC * ACC_ROWS, DW), jnp.float32),
    mesh=_MESH,
    scratch_types=[
        pltpu.VMEM((CHUNK,), jnp.int32),
        pltpu.VMEM((CHUNK, DW), jnp.float32),
        pltpu.VMEM_SHARED((ACC_ROWS, DW), jnp.float32),
    ],
)
def _sc_degree(dst_hbm, ones_hbm, zeros_hbm, out_hbm, idx_v, ones_v, acc):
    c = lax.axis_index("c")
    s = lax.axis_index("s")
    wid = s * NC + c
    pltpu.sync_copy(zeros_hbm, acc.at[pl.ds(s * RPT, RPT)])
    pltpu.sync_copy(ones_hbm, ones_v)
    plsc.subcore_barrier()
    base = wid * EPW

    def body(j, carry):
        pltpu.sync_copy(dst_hbm.at[pl.ds(base + j * CHUNK, CHUNK)], idx_v)
        pltpu.sync_copy(ones_v, acc.at[idx_v], add=True)
        return carry

    lax.fori_loop(0, CPW, body, 0)
    plsc.subcore_barrier()
    pltpu.sync_copy(
        acc.at[pl.ds(s * RPT, RPT)],
        out_hbm.at[pl.ds(c * ACC_ROWS + s * RPT, RPT)],
    )


@functools.partial(
    pl.kernel,
    out_type=jax.ShapeDtypeStruct((NC * ACC_ROWS, W128), jnp.float32),
    mesh=_MESH,
    scratch_types=[
        pltpu.VMEM((CHUNK,), jnp.int32),
        pltpu.VMEM((CHUNK,), jnp.int32),
        pltpu.VMEM((CHUNK, W128), jnp.float32),
        pltpu.VMEM_SHARED((ACC_ROWS, W128), jnp.float32),
        pltpu.SemaphoreType.DMA,
    ],
)
def _sc_agg(h_hbm, src_hbm, dst_hbm, zeros_hbm, out_hbm,
            sidx, didx, rows, acc, semg):
    c = lax.axis_index("c")
    s = lax.axis_index("s")
    wid = s * NC + c
    pltpu.sync_copy(zeros_hbm, acc.at[pl.ds(s * RPT, RPT)])
    plsc.subcore_barrier()
    base = wid * EPW

    def body(j, carry):
        start = base + j * CHUNK
        pltpu.sync_copy(src_hbm.at[pl.ds(start, CHUNK)], sidx)
        g = pltpu.async_copy(h_hbm.at[sidx], rows, semg)
        pltpu.sync_copy(dst_hbm.at[pl.ds(start, CHUNK)], didx)
        g.wait()
        pltpu.sync_copy(rows, acc.at[didx], add=True)
        return carry

    lax.fori_loop(0, CPW, body, 0)
    plsc.subcore_barrier()
    pltpu.sync_copy(
        acc.at[pl.ds(s * RPT, RPT)],
        out_hbm.at[pl.ds(c * ACC_ROWS + s * RPT, RPT)],
    )


def _dinv_of(degp_ref):
    deg = degp_ref[0, :, :1] + degp_ref[1, :, :1] + 1.0
    return lax.rsqrt(deg)


def _mm_prescale_body(x_ref, w_ref, degp_ref, out_ref):
    h = jnp.dot(x_ref[...], w_ref[...], preferred_element_type=jnp.float32)
    out_ref[...] = h * _dinv_of(degp_ref)


def _mid_body(aggp_ref, hp_ref, degp_ref, b_ref, w_ref, out_ref):
    dinv = _dinv_of(degp_ref)
    u = (aggp_ref[0] + aggp_ref[1] + hp_ref[...]) * dinv + b_ref[...]
    t = jnp.maximum(u, 0.0)
    h = jnp.dot(t, w_ref[...], preferred_element_type=jnp.float32)
    out_ref[...] = h * dinv


def _tail_body(aggp_ref, hp_ref, degp_ref, b2_ref, wh1_ref, bh1_ref,
               wh2_ref, bh2_ref, out_ref):
    dinv = _dinv_of(degp_ref)
    u = (aggp_ref[0] + aggp_ref[1] + hp_ref[...]) * dinv + b2_ref[...]
    g = jnp.maximum(u, 0.0)
    t = jnp.dot(g, wh1_ref[...], preferred_element_type=jnp.float32)
    t = jnp.maximum(t + bh1_ref[...], 0.0)
    out_ref[...] = jnp.dot(t, wh2_ref[...], preferred_element_type=jnp.float32) + bh2_ref[...]


def _row_spec(width):
    return pl.BlockSpec((MB, width), lambda i: (i, 0))


def _full_spec(shape):
    nd = len(shape)
    return pl.BlockSpec(shape, lambda i: (0,) * nd)


_PART_SPEC = pl.BlockSpec((2, MB, W128), lambda i: (0, i, 0))
_DEG_SPEC = pl.BlockSpec((2, MB, DW), lambda i: (0, i, 0))

_GRID = (N // MB,)

_mm_prescale = pl.pallas_call(
    _mm_prescale_body,
    grid=_GRID,
    in_specs=[_row_spec(IN_CH), _full_spec((IN_CH, W128)), _DEG_SPEC],
    out_specs=_row_spec(W128),
    out_shape=jax.ShapeDtypeStruct((N, W128), jnp.float32),
)

_mid = pl.pallas_call(
    _mid_body,
    grid=_GRID,
    in_specs=[_PART_SPEC, _row_spec(W128), _DEG_SPEC,
              _full_spec((1, W128)), _full_spec((W128, W128))],
    out_specs=_row_spec(W128),
    out_shape=jax.ShapeDtypeStruct((N, W128), jnp.float32),
)

_tail = pl.pallas_call(
    _tail_body,
    grid=_GRID,
    in_specs=[_PART_SPEC, _row_spec(W128), _DEG_SPEC,
              _full_spec((1, W128)), _full_spec((W128, W128)),
              _full_spec((1, W128)), _full_spec((W128, W128)),
              _full_spec((1, W128))],
    out_specs=_row_spec(W128),
    out_shape=jax.ShapeDtypeStruct((N, W128), jnp.float32),
)


def _padw(a, rows=None):
    r = a.shape[0] if rows is None else rows
    out = jnp.zeros((r, W128), jnp.float32)
    return out.at[: a.shape[0], : a.shape[1]].set(a)


def kernel(x, edge_index, W1, b1, W2, b2, Wh1, bh1, Wh2, bh2):
    ei = edge_index.astype(jnp.int32)
    pad = EPAD - E
    src_p = jnp.concatenate([ei[0], jnp.zeros((pad,), jnp.int32)])
    dst_p = jnp.concatenate([ei[1], jnp.full((pad,), N, jnp.int32)])

    ones_d = jnp.ones((CHUNK, DW), jnp.float32)
    zeros128 = jnp.zeros((RPT, W128), jnp.float32)

    degp = _sc_degree(dst_p, ones_d, zeros128).reshape(NC, ACC_ROWS, DW)

    h1p = _mm_prescale(x, _padw(W1), degp)
    agg1 = _sc_agg(h1p, src_p, dst_p, zeros128).reshape(NC, ACC_ROWS, W128)

    h2p = _mid(agg1, h1p, degp, _padw(b1.reshape(1, HID)), _padw(W2, W128))
    agg2 = _sc_agg(h2p, src_p, dst_p, zeros128).reshape(NC, ACC_ROWS, W128)

    y = _tail(agg2, h2p, degp, _padw(b2.reshape(1, HID)),
              _padw(Wh1, W128), _padw(bh1.reshape(1, HID // 2)),
              _padw(Wh2, W128), _padw(bh2.reshape(1, 2)))
    return y[:, :2]


# speedup vs baseline: 1.0593x; 1.0593x over previous
import functools

import jax
import jax.numpy as jnp
from jax import lax
from jax.experimental import pallas as pl
from jax.experimental.pallas import tpu as pltpu
from jax.experimental.pallas import tpu_sc as plsc

N = 10000
E = 320000
IN_CH = 128
HID = 64
W128 = 128
DW = 128

NC = 2
NS = 16
NW = NC * NS
CHUNK = 128
CPW = 80
EPW = CPW * CHUNK
EPAD = NW * EPW
RPT = 640
ACC_ROWS = NS * RPT
MB = 1000

_MESH = plsc.VectorSubcoreMesh(
    core_axis_name="c", subcore_axis_name="s", num_cores=NC, num_subcores=NS
)


@functools.partial(
    pl.kernel,
    out_type=jax.ShapeDtypeStruct((NC * ACC_ROWS, DW), jnp.float32),
    mesh=_MESH,
    scratch_types=[
        pltpu.VMEM((CHUNK,), jnp.int32),
        pltpu.VMEM((CHUNK, DW), jnp.float32),
        pltpu.VMEM_SHARED((ACC_ROWS, DW), jnp.float32),
    ],
)
def _sc_degree(dst_hbm, ones_hbm, zeros_hbm, out_hbm, idx_v, ones_v, acc):
    c = lax.axis_index("c")
    s = lax.axis_index("s")
    wid = s * NC + c
    pltpu.sync_copy(zeros_hbm, acc.at[pl.ds(s * RPT, RPT)])
    pltpu.sync_copy(ones_hbm, ones_v)
    plsc.subcore_barrier()
    base = wid * EPW

    def body(j, carry):
        pltpu.sync_copy(dst_hbm.at[pl.ds(base + j * CHUNK, CHUNK)], idx_v)
        pltpu.sync_copy(ones_v, acc.at[idx_v], add=True)
        return carry

    lax.fori_loop(0, CPW, body, 0)
    plsc.subcore_barrier()
    pltpu.sync_copy(
        acc.at[pl.ds(s * RPT, RPT)],
        out_hbm.at[pl.ds(c * ACC_ROWS + s * RPT, RPT)],
    )


@functools.partial(
    pl.kernel,
    out_type=jax.ShapeDtypeStruct((NC * ACC_ROWS, W128), jnp.float32),
    mesh=_MESH,
    scratch_types=[
        pltpu.VMEM((CHUNK,), jnp.int32),
        pltpu.VMEM((CHUNK,), jnp.int32),
        pltpu.VMEM((CHUNK,), jnp.int32),
        pltpu.VMEM((CHUNK,), jnp.int32),
        pltpu.VMEM((CHUNK, W128), jnp.float32),
        pltpu.VMEM((CHUNK, W128), jnp.float32),
        pltpu.VMEM_SHARED((ACC_ROWS, W128), jnp.float32),
        pltpu.SemaphoreType.DMA,
        pltpu.SemaphoreType.DMA,
        pltpu.SemaphoreType.DMA,
        pltpu.SemaphoreType.DMA,
        pltpu.SemaphoreType.DMA,
        pltpu.SemaphoreType.DMA,
    ],
)
def _sc_agg(h_hbm, src_hbm, dst_hbm, zeros_hbm, out_hbm,
            sidx0, sidx1, didx0, didx1, rows0, rows1, acc,
            semi0, semi1, semg0, semg1, sems0, sems1):
    c = lax.axis_index("c")
    s = lax.axis_index("s")
    wid = s * NC + c
    pltpu.sync_copy(zeros_hbm, acc.at[pl.ds(s * RPT, RPT)])
    plsc.subcore_barrier()
    base = wid * EPW

    def issue_idx(j, si, di, sem):
        start = base + j * CHUNK
        pltpu.async_copy(src_hbm.at[pl.ds(start, CHUNK)], si, sem)
        pltpu.async_copy(dst_hbm.at[pl.ds(start, CHUNK)], di, sem)

    def drain_idx(si, di, sem):
        pltpu.make_async_copy(src_hbm.at[pl.ds(0, CHUNK)], si, sem).wait()
        pltpu.make_async_copy(dst_hbm.at[pl.ds(0, CHUNK)], di, sem).wait()

    issue_idx(0, sidx0, didx0, semi0)
    issue_idx(1, sidx1, didx1, semi1)

    def body(i, carry):
        drain_idx(sidx0, didx0, semi0)
        g0 = pltpu.async_copy(h_hbm.at[sidx0], rows0, semg0)
        drain_idx(sidx1, didx1, semi1)
        g1 = pltpu.async_copy(h_hbm.at[sidx1], rows1, semg1)
        g0.wait()
        s0 = pltpu.async_copy(rows0, acc.at[didx0], sems0, add=True)
        g1.wait()
        s1 = pltpu.async_copy(rows1, acc.at[didx1], sems1, add=True)
        s0.wait()
        s1.wait()

        @pl.when(i < CPW // 2 - 1)
        def _():
            issue_idx(2 * i + 2, sidx0, didx0, semi0)
            issue_idx(2 * i + 3, sidx1, didx1, semi1)

        return carry

    lax.fori_loop(0, CPW // 2, body, 0)
    plsc.subcore_barrier()
    pltpu.sync_copy(
        acc.at[pl.ds(s * RPT, RPT)],
        out_hbm.at[pl.ds(c * ACC_ROWS + s * RPT, RPT)],
    )


def _dinv_of(degp_ref):
    deg = degp_ref[0, :, :1] + degp_ref[1, :, :1] + 1.0
    return lax.rsqrt(deg)


def _mm_prescale_body(x_ref, w_ref, degp_ref, out_ref):
    h = jnp.dot(x_ref[...], w_ref[...], preferred_element_type=jnp.float32)
    out_ref[...] = h * _dinv_of(degp_ref)


def _mid_body(aggp_ref, hp_ref, degp_ref, b_ref, w_ref, out_ref):
    dinv = _dinv_of(degp_ref)
    u = (aggp_ref[0] + aggp_ref[1] + hp_ref[...]) * dinv + b_ref[...]
    t = jnp.maximum(u, 0.0)
    h = jnp.dot(t, w_ref[...], preferred_element_type=jnp.float32)
    out_ref[...] = h * dinv


def _tail_body(aggp_ref, hp_ref, degp_ref, b2_ref, wh1_ref, bh1_ref,
               wh2_ref, bh2_ref, out_ref):
    dinv = _dinv_of(degp_ref)
    u = (aggp_ref[0] + aggp_ref[1] + hp_ref[...]) * dinv + b2_ref[...]
    g = jnp.maximum(u, 0.0)
    t = jnp.dot(g, wh1_ref[...], preferred_element_type=jnp.float32)
    t = jnp.maximum(t + bh1_ref[...], 0.0)
    out_ref[...] = jnp.dot(t, wh2_ref[...], preferred_element_type=jnp.float32) + bh2_ref[...]


def _row_spec(width):
    return pl.BlockSpec((MB, width), lambda i: (i, 0))


def _full_spec(shape):
    nd = len(shape)
    return pl.BlockSpec(shape, lambda i: (0,) * nd)


_PART_SPEC = pl.BlockSpec((2, MB, W128), lambda i: (0, i, 0))
_DEG_SPEC = pl.BlockSpec((2, MB, DW), lambda i: (0, i, 0))

_GRID = (N // MB,)

_mm_prescale = pl.pallas_call(
    _mm_prescale_body,
    grid=_GRID,
    in_specs=[_row_spec(IN_CH), _full_spec((IN_CH, W128)), _DEG_SPEC],
    out_specs=_row_spec(W128),
    out_shape=jax.ShapeDtypeStruct((N, W128), jnp.float32),
)

_mid = pl.pallas_call(
    _mid_body,
    grid=_GRID,
    in_specs=[_PART_SPEC, _row_spec(W128), _DEG_SPEC,
              _full_spec((1, W128)), _full_spec((W128, W128))],
    out_specs=_row_spec(W128),
    out_shape=jax.ShapeDtypeStruct((N, W128), jnp.float32),
)

_tail = pl.pallas_call(
    _tail_body,
    grid=_GRID,
    in_specs=[_PART_SPEC, _row_spec(W128), _DEG_SPEC,
              _full_spec((1, W128)), _full_spec((W128, W128)),
              _full_spec((1, W128)), _full_spec((W128, W128)),
              _full_spec((1, W128))],
    out_specs=_row_spec(W128),
    out_shape=jax.ShapeDtypeStruct((N, W128), jnp.float32),
)


def _padw(a, rows=None):
    r = a.shape[0] if rows is None else rows
    out = jnp.zeros((r, W128), jnp.float32)
    return out.at[: a.shape[0], : a.shape[1]].set(a)


def kernel(x, edge_index, W1, b1, W2, b2, Wh1, bh1, Wh2, bh2):
    ei = edge_index.astype(jnp.int32)
    pad = EPAD - E
    src_p = jnp.concatenate([ei[0], jnp.zeros((pad,), jnp.int32)])
    dst_p = jnp.concatenate([ei[1], jnp.full((pad,), N, jnp.int32)])

    ones_d = jnp.ones((CHUNK, DW), jnp.float32)
    zeros128 = jnp.zeros((RPT, W128), jnp.float32)

    degp = _sc_degree(dst_p, ones_d, zeros128).reshape(NC, ACC_ROWS, DW)

    h1p = _mm_prescale(x, _padw(W1), degp)
    agg1 = _sc_agg(h1p, src_p, dst_p, zeros128).reshape(NC, ACC_ROWS, W128)

    h2p = _mid(agg1, h1p, degp, _padw(b1.reshape(1, HID)), _padw(W2, W128))
    agg2 = _sc_agg(h2p, src_p, dst_p, zeros128).reshape(NC, ACC_ROWS, W128)

    y = _tail(agg2, h2p, degp, _padw(b2.reshape(1, HID)),
              _padw(Wh1, W128), _padw(bh1.reshape(1, HID // 2)),
              _padw(Wh2, W128), _padw(bh2.reshape(1, 2)))
    return y[:, :2]


# speedup vs baseline: 1.0598x; 1.0004x over previous
import functools

import jax
import jax.numpy as jnp
from jax import lax
from jax.experimental import pallas as pl
from jax.experimental.pallas import tpu as pltpu
from jax.experimental.pallas import tpu_sc as plsc

N = 10000
E = 320000
IN_CH = 128
HID = 64
W128 = 128
DW = 128

NC = 2
NS = 16
NW = NC * NS
CHUNK = 128
CPW = 80
EPW = CPW * CHUNK
EPAD = NW * EPW
RPT = 640
ACC_ROWS = NS * RPT
MB = 1000


@functools.cache
def _sc_kernels():
    mesh = plsc.VectorSubcoreMesh(
        core_axis_name="c", subcore_axis_name="s", num_cores=NC, num_subcores=NS
    )
    return (_build_sc_degree(mesh), _build_sc_agg(mesh))


def _build_sc_degree(_MESH):
    return functools.partial(
        pl.kernel,
        out_type=jax.ShapeDtypeStruct((NC * ACC_ROWS, DW), jnp.float32),
        mesh=_MESH,
        scratch_types=[
            pltpu.VMEM((CHUNK,), jnp.int32),
            pltpu.VMEM((CHUNK, DW), jnp.float32),
            pltpu.VMEM_SHARED((ACC_ROWS, DW), jnp.float32),
        ],
    )(_sc_degree_body)


def _sc_degree_body(dst_hbm, ones_hbm, zeros_hbm, out_hbm, idx_v, ones_v, acc):
    c = lax.axis_index("c")
    s = lax.axis_index("s")
    wid = s * NC + c
    pltpu.sync_copy(zeros_hbm, acc.at[pl.ds(s * RPT, RPT)])
    pltpu.sync_copy(ones_hbm, ones_v)
    plsc.subcore_barrier()
    base = wid * EPW

    def body(j, carry):
        pltpu.sync_copy(dst_hbm.at[pl.ds(base + j * CHUNK, CHUNK)], idx_v)
        pltpu.sync_copy(ones_v, acc.at[idx_v], add=True)
        return carry

    lax.fori_loop(0, CPW, body, 0)
    plsc.subcore_barrier()
    pltpu.sync_copy(
        acc.at[pl.ds(s * RPT, RPT)],
        out_hbm.at[pl.ds(c * ACC_ROWS + s * RPT, RPT)],
    )


def _build_sc_agg(_MESH):
    return functools.partial(
        pl.kernel,
        out_type=jax.ShapeDtypeStruct((NC * ACC_ROWS, W128), jnp.float32),
        mesh=_MESH,
        scratch_types=[
            pltpu.VMEM((CHUNK,), jnp.int32),
            pltpu.VMEM((CHUNK,), jnp.int32),
            pltpu.VMEM((CHUNK,), jnp.int32),
            pltpu.VMEM((CHUNK,), jnp.int32),
            pltpu.VMEM((CHUNK, W128), jnp.float32),
            pltpu.VMEM((CHUNK, W128), jnp.float32),
            pltpu.VMEM_SHARED((ACC_ROWS, W128), jnp.float32),
            pltpu.SemaphoreType.DMA,
            pltpu.SemaphoreType.DMA,
            pltpu.SemaphoreType.DMA,
            pltpu.SemaphoreType.DMA,
            pltpu.SemaphoreType.DMA,
            pltpu.SemaphoreType.DMA,
        ],
    )(_sc_agg_body)


def _sc_agg_body(h_hbm, src_hbm, dst_hbm, zeros_hbm, out_hbm,
                 sidx0, sidx1, didx0, didx1, rows0, rows1, acc,
                 semi0, semi1, semg0, semg1, sems0, sems1):
    c = lax.axis_index("c")
    s = lax.axis_index("s")
    wid = s * NC + c
    pltpu.sync_copy(zeros_hbm, acc.at[pl.ds(s * RPT, RPT)])
    plsc.subcore_barrier()
    base = wid * EPW

    def issue_idx(j, si, di, sem):
        start = base + j * CHUNK
        pltpu.async_copy(src_hbm.at[pl.ds(start, CHUNK)], si, sem)
        pltpu.async_copy(dst_hbm.at[pl.ds(start, CHUNK)], di, sem)

    def drain_idx(si, di, sem):
        pltpu.make_async_copy(src_hbm.at[pl.ds(0, CHUNK)], si, sem).wait()
        pltpu.make_async_copy(dst_hbm.at[pl.ds(0, CHUNK)], di, sem).wait()

    issue_idx(0, sidx0, didx0, semi0)
    issue_idx(1, sidx1, didx1, semi1)

    def body(i, carry):
        drain_idx(sidx0, didx0, semi0)
        g0 = pltpu.async_copy(h_hbm.at[sidx0], rows0, semg0)
        drain_idx(sidx1, didx1, semi1)
        g1 = pltpu.async_copy(h_hbm.at[sidx1], rows1, semg1)
        g0.wait()
        s0 = pltpu.async_copy(rows0, acc.at[didx0], sems0, add=True)
        g1.wait()
        s1 = pltpu.async_copy(rows1, acc.at[didx1], sems1, add=True)
        s0.wait()
        s1.wait()

        @pl.when(i < CPW // 2 - 1)
        def _():
            issue_idx(2 * i + 2, sidx0, didx0, semi0)
            issue_idx(2 * i + 3, sidx1, didx1, semi1)

        return carry

    lax.fori_loop(0, CPW // 2, body, 0)
    plsc.subcore_barrier()
    pltpu.sync_copy(
        acc.at[pl.ds(s * RPT, RPT)],
        out_hbm.at[pl.ds(c * ACC_ROWS + s * RPT, RPT)],
    )


def _dinv_of(degp_ref):
    deg = degp_ref[0, :, :1] + degp_ref[1, :, :1] + 1.0
    return lax.rsqrt(deg)


def _mm_prescale_body(x_ref, w_ref, degp_ref, out_ref):
    h = jnp.dot(x_ref[...], w_ref[...], preferred_element_type=jnp.float32)
    out_ref[...] = h * _dinv_of(degp_ref)


def _mid_body(aggp_ref, hp_ref, degp_ref, b_ref, w_ref, out_ref):
    dinv = _dinv_of(degp_ref)
    u = (aggp_ref[0] + aggp_ref[1] + hp_ref[...]) * dinv + b_ref[...]
    t = jnp.maximum(u, 0.0)
    h = jnp.dot(t, w_ref[...], preferred_element_type=jnp.float32)
    out_ref[...] = h * dinv


def _tail_body(aggp_ref, hp_ref, degp_ref, b2_ref, wh1_ref, bh1_ref,
               wh2_ref, bh2_ref, out_ref):
    dinv = _dinv_of(degp_ref)
    u = (aggp_ref[0] + aggp_ref[1] + hp_ref[...]) * dinv + b2_ref[...]
    g = jnp.maximum(u, 0.0)
    t = jnp.dot(g, wh1_ref[...], preferred_element_type=jnp.float32)
    t = jnp.maximum(t + bh1_ref[...], 0.0)
    out_ref[...] = jnp.dot(t, wh2_ref[...], preferred_element_type=jnp.float32) + bh2_ref[...]


def _row_spec(width):
    return pl.BlockSpec((MB, width), lambda i: (i, 0))


def _full_spec(shape):
    nd = len(shape)
    return pl.BlockSpec(shape, lambda i: (0,) * nd)


_PART_SPEC = pl.BlockSpec((2, MB, W128), lambda i: (0, i, 0))
_DEG_SPEC = pl.BlockSpec((2, MB, DW), lambda i: (0, i, 0))

_GRID = (N // MB,)

_mm_prescale = pl.pallas_call(
    _mm_prescale_body,
    grid=_GRID,
    in_specs=[_row_spec(IN_CH), _full_spec((IN_CH, W128)), _DEG_SPEC],
    out_specs=_row_spec(W128),
    out_shape=jax.ShapeDtypeStruct((N, W128), jnp.float32),
)

_mid = pl.pallas_call(
    _mid_body,
    grid=_GRID,
    in_specs=[_PART_SPEC, _row_spec(W128), _DEG_SPEC,
              _full_spec((1, W128)), _full_spec((W128, W128))],
    out_specs=_row_spec(W128),
    out_shape=jax.ShapeDtypeStruct((N, W128), jnp.float32),
)

_tail = pl.pallas_call(
    _tail_body,
    grid=_GRID,
    in_specs=[_PART_SPEC, _row_spec(W128), _DEG_SPEC,
              _full_spec((1, W128)), _full_spec((W128, W128)),
              _full_spec((1, W128)), _full_spec((W128, W128)),
              _full_spec((1, W128))],
    out_specs=_row_spec(W128),
    out_shape=jax.ShapeDtypeStruct((N, W128), jnp.float32),
)


def _padw(a, rows=None):
    r = a.shape[0] if rows is None else rows
    out = jnp.zeros((r, W128), jnp.float32)
    return out.at[: a.shape[0], : a.shape[1]].set(a)


def kernel(x, edge_index, W1, b1, W2, b2, Wh1, bh1, Wh2, bh2):
    ei = edge_index.astype(jnp.int32)
    pad = EPAD - E
    src_p = jnp.concatenate([ei[0], jnp.zeros((pad,), jnp.int32)])
    dst_p = jnp.concatenate([ei[1], jnp.full((pad,), N, jnp.int32)])

    ones_d = jnp.ones((CHUNK, DW), jnp.float32)
    zeros128 = jnp.zeros((RPT, W128), jnp.float32)

    _sc_degree, _sc_agg = _sc_kernels()
    degp = _sc_degree(dst_p, ones_d, zeros128).reshape(NC, ACC_ROWS, DW)

    h1p = _mm_prescale(x, _padw(W1), degp)
    agg1 = _sc_agg(h1p, src_p, dst_p, zeros128).reshape(NC, ACC_ROWS, W128)

    h2p = _mid(agg1, h1p, degp, _padw(b1.reshape(1, HID)), _padw(W2, W128))
    agg2 = _sc_agg(h2p, src_p, dst_p, zeros128).reshape(NC, ACC_ROWS, W128)

    y = _tail(agg2, h2p, degp, _padw(b2.reshape(1, HID)),
              _padw(Wh1, W128), _padw(bh1.reshape(1, HID // 2)),
              _padw(Wh2, W128), _padw(bh2.reshape(1, 2)))
    return y[:, :2]
